# Initial kernel scaffold; baseline (speedup 1.0000x reference)
import jax
import jax.numpy as jnp
from jax.experimental import pallas as pl


def kernel(at_no, coord, edge_index, charge, spin, embed_table, atom_sp, mw1, mb1, mw2, mb2, fw, fb, uu, uv, uw1, ub1, uw2, ub2, ow1, ob1, ow2, ob2):
    raise NotImplementedError("write your pallas kernel here")


# speedup vs baseline: 5.6427x; 5.6427x over previous
import functools

import jax
import jax.numpy as jnp
from jax import lax
from jax.experimental import pallas as pl
from jax.experimental.pallas import tpu as pltpu
from jax.experimental.pallas import tpu_sc as plsc

_NC, _NS = 2, 16
_NW = _NC * _NS
_CH = 128
_CUTOFF = 5.0
_NB = 20


@functools.lru_cache(maxsize=None)
def _make_gather(n_rows, feat, batch):
    assert batch % (_NW * _CH) == 0 and feat % 128 == 0
    per_w = batch // _NW
    n_chunks = per_w // _CH
    mesh = plsc.VectorSubcoreMesh(core_axis_name="c", subcore_axis_name="s")

    @functools.partial(
        pl.kernel,
        out_type=jax.ShapeDtypeStruct((batch, feat), jnp.float32),
        mesh=mesh,
        scratch_types=[
            pltpu.VMEM((_CH,), jnp.int32),
            pltpu.VMEM((_CH, feat), jnp.float32),
            pltpu.SemaphoreType.DMA,
        ],
    )
    def gk(table, idx, out, idx_v, rows_v, sem):
        wid = lax.axis_index("s") * _NC + lax.axis_index("c")
        base = pl.multiple_of(wid * per_w, _CH)

        def body(j, carry):
            off = pl.multiple_of(base + j * _CH, _CH)
            pltpu.sync_copy(idx.at[pl.ds(off, _CH)], idx_v)
            pltpu.async_copy(table.at[idx_v], rows_v, sem).wait()
            pltpu.sync_copy(rows_v, out.at[pl.ds(off, _CH)])
            return carry

        lax.fori_loop(0, n_chunks, body, 0)

    return gk


@functools.lru_cache(maxsize=None)
def _make_scatter(n_rows, feat, batch):
    assert batch % (_NW * _CH) == 0 and feat % 128 == 0
    assert n_rows % (_NS * _CH) == 0
    per_w = batch // _NW
    n_chunks = per_w // _CH
    rpt = n_rows // _NS
    mesh = plsc.VectorSubcoreMesh(core_axis_name="c", subcore_axis_name="s")

    @functools.partial(
        pl.kernel,
        out_type=jax.ShapeDtypeStruct((_NC * n_rows, feat), jnp.float32),
        mesh=mesh,
        scratch_types=[
            pltpu.VMEM((_CH,), jnp.int32),
            pltpu.VMEM((_CH, feat), jnp.float32),
            pltpu.VMEM_SHARED((n_rows, feat), jnp.float32),
        ],
    )
    def sk(upd, idx, zeros, out, idx_v, rows_v, acc):
        c = lax.axis_index("c")
        s = lax.axis_index("s")

        def zcopy(q, carry):
            pltpu.sync_copy(zeros, acc.at[pl.ds(s * rpt + q * _CH, _CH)])
            return carry

        lax.fori_loop(0, rpt // _CH, zcopy, 0)
        plsc.subcore_barrier()

        base = pl.multiple_of((c * _NS + s) * per_w, _CH)

        def body(j, carry):
            off = pl.multiple_of(base + j * _CH, _CH)
            pltpu.sync_copy(idx.at[pl.ds(off, _CH)], idx_v)
            pltpu.sync_copy(upd.at[pl.ds(off, _CH)], rows_v)
            pltpu.sync_copy(rows_v, acc.at[idx_v], add=True)
            return carry

        lax.fori_loop(0, n_chunks, body, 0)
        plsc.subcore_barrier()
        dst_row = pl.multiple_of(c * n_rows + s * rpt, _CH)
        pltpu.sync_copy(acc.at[pl.ds(s * rpt, rpt)], out.at[pl.ds(dst_row, rpt)])

    return sk


_NROWS_PAD = _NS * _CH


@jax.custom_vjp
def _gather(table, idx):
    return _make_gather(table.shape[0], table.shape[1], idx.shape[0])(table, idx)


def _gather_fwd(table, idx):
    return _gather(table, idx), (idx, table.shape[0])


def _gather_bwd(res, ct):
    idx, n = res
    return _scatter(ct, idx, n), None


@functools.partial(jax.custom_vjp, nondiff_argnums=(2,))
def _scatter(upd, idx, n):
    assert n % _NROWS_PAD == 0
    zeros = jnp.zeros((_CH, upd.shape[1]), jnp.float32)
    parts = _make_scatter(n, upd.shape[1], upd.shape[0])(upd, idx, zeros)
    return parts[:n] + parts[n:]


def _scatter_fwd(upd, idx, n):
    return _scatter(upd, idx, n), idx


def _scatter_bwd(n, res, ct):
    idx = res
    return _gather(ct, idx), None


_gather.defvjp(_gather_fwd, _gather_bwd)
_scatter.defvjp(_scatter_fwd, _scatter_bwd)


def kernel(at_no, coord, edge_index, charge, spin, embed_table, atom_sp, mw1, mb1, mw2, mb2, fw, fb, uu, uv, uw1, ub1, uw2, ub2, ow1, ob1, ow2, ob2):
    N, D = coord.shape[0], embed_table.shape[1]
    E = edge_index.shape[1]
    L = mw1.shape[0]
    nelem = embed_table.shape[0]
    src, dst = edge_index[0], edge_index[1]

    quant = _NW * _CH
    Ep = ((E + quant - 1) // quant) * quant
    Np = ((N + quant - 1) // quant) * quant
    src_p = jnp.concatenate([src, jnp.zeros((Ep - E,), src.dtype)])
    dst_p = jnp.concatenate([dst, jnp.zeros((Ep - E,), dst.dtype)])
    at_no_p = jnp.concatenate([at_no, jnp.zeros((Np - N,), at_no.dtype)])
    emask = (jnp.arange(Ep) < E).astype(jnp.float32)

    NP = ((N + _NROWS_PAD - 1) // _NROWS_PAD) * _NROWS_PAD

    nelem_p = ((nelem + 127) // 128) * 128
    tbl = jnp.concatenate(
        [embed_table, atom_sp[:, None], jnp.zeros((nelem, 127), jnp.float32)], axis=1)
    tbl = jnp.pad(tbl, ((0, nelem_p - nelem), (0, 0)))
    g0 = _gather(tbl, at_no_p)
    x_s0 = jnp.pad(g0[:N, :D], ((0, NP - N), (0, 0)))
    sp_sum = jnp.sum(g0[:N, D])

    coordp0 = jnp.pad(coord, ((0, NP - N), (0, 125)))
    sd_p = jnp.concatenate([src_p, dst_p])

    centers = jnp.linspace(0.0, _CUTOFF, _NB)
    gamma = (_NB / _CUTOFF) ** 2

    def efn(coordp):
        cg = _gather(coordp, sd_p)
        vec = cg[:Ep, :3] - cg[Ep:, :3]
        dist = jnp.sqrt(jnp.sum(vec * vec, axis=-1) + 1e-9)
        dirv = vec / dist[:, None]
        rbf = jnp.exp(-gamma * (dist[:, None] - centers[None, :]) ** 2)
        fcut = 0.5 * (jnp.cos(jnp.pi * jnp.clip(dist, 0.0, _CUTOFF) / _CUTOFF) + 1.0) * emask
        x_s = x_s0
        xv = [jnp.zeros((NP, D), jnp.float32) for _ in range(3)]
        for l in range(L):
            phi = jax.nn.silu(x_s @ mw1[l] + mb1[l]) @ mw2[l] + mb2[l]
            Wf = (rbf @ fw[l] + fb[l]) * fcut[:, None]
            p0 = _gather(phi[:, :D], src_p)
            p1 = _gather(phi[:, D:2 * D], src_p)
            p2 = _gather(phi[:, 2 * D:], src_p)
            ms = p0 * Wf[:, :D]
            mv1 = p1 * Wf[:, D:2 * D]
            mv2 = p2 * Wf[:, 2 * D:]
            x_s = x_s + _scatter(ms, dst_p, NP)
            xv = [xv[k] + _scatter(_gather(xv[k], src_p) * mv1 + dirv[:, k:k + 1] * mv2, dst_p, NP)
                  for k in range(3)]
            xvs = jnp.stack(xv, axis=1)
            Uv = xvs @ uu[l]
            Vv = xvs @ uv[l]
            Vn = jnp.sqrt(jnp.sum(Vv * Vv, axis=1) + 1e-9)
            a = jax.nn.silu(jnp.concatenate([x_s, Vn], axis=-1) @ uw1[l] + ub1[l]) @ uw2[l] + ub2[l]
            avv, asv, ass = jnp.split(a, 3, axis=-1)
            xvs = xvs + avv[:, None, :] * Uv
            x_s = x_s + ass + asv * jnp.sum(Uv * Vv, axis=1)
            xv = [xvs[:, 0], xvs[:, 1], xvs[:, 2]]
        atom_out = jax.nn.silu(x_s @ ow1 + ob1) @ ow2 + ob2
        return jnp.sum(atom_out[:N]) + sp_sum

    energy, gpad = jax.value_and_grad(efn)(coordp0)
    return (energy, gpad[:N, :3])


# speedup vs baseline: 6.2150x; 1.1014x over previous
import functools

import jax
import jax.numpy as jnp
from jax import lax
from jax.experimental import pallas as pl
from jax.experimental.pallas import tpu as pltpu
from jax.experimental.pallas import tpu_sc as plsc

_NC, _NS = 2, 16
_NW = _NC * _NS
_CH = 128
_CUTOFF = 5.0
_NB = 20


@functools.lru_cache(maxsize=None)
def _make_gather(n_rows, feat, batch):
    assert batch % (_NW * _CH * 2) == 0 and feat % 128 == 0
    per_w = batch // _NW
    n_chunks = per_w // _CH
    mesh = plsc.VectorSubcoreMesh(core_axis_name="c", subcore_axis_name="s")

    @functools.partial(
        pl.kernel,
        out_type=jax.ShapeDtypeStruct((batch, feat), jnp.float32),
        mesh=mesh,
        scratch_types=[
            pltpu.VMEM((per_w,), jnp.int32),
            pltpu.VMEM((_CH, feat), jnp.float32),
            pltpu.VMEM((_CH, feat), jnp.float32),
            pltpu.SemaphoreType.DMA,
            pltpu.SemaphoreType.DMA,
        ],
    )
    def gk(table, idx, out, idx_full, rows0, rows1, sem0, sem1):
        wid = lax.axis_index("s") * _NC + lax.axis_index("c")
        base = pl.multiple_of(wid * per_w, _CH)
        pltpu.sync_copy(idx.at[pl.ds(base, per_w)], idx_full)

        def ichunk(j):
            return idx_full.at[pl.ds(j * _CH, _CH)]

        pltpu.async_copy(table.at[ichunk(0)], rows0, sem0)

        def body(p, carry):
            j0 = p * 2
            pltpu.async_copy(table.at[ichunk(j0 + 1)], rows1, sem1)
            pltpu.make_async_copy(table.at[ichunk(j0)], rows0, sem0).wait()
            pltpu.sync_copy(rows0, out.at[pl.ds(base + j0 * _CH, _CH)])

            @pl.when(p + 1 < n_chunks // 2)
            def _():
                pltpu.async_copy(table.at[ichunk(j0 + 2)], rows0, sem0)

            pltpu.make_async_copy(table.at[ichunk(j0 + 1)], rows1, sem1).wait()
            pltpu.sync_copy(rows1, out.at[pl.ds(base + (j0 + 1) * _CH, _CH)])
            return carry

        lax.fori_loop(0, n_chunks // 2, body, 0)

    return gk


@functools.lru_cache(maxsize=None)
def _make_scatter(n_rows, feat, batch):
    assert batch % (_NW * _CH) == 0 and feat % 128 == 0
    assert n_rows % (_NS * _CH) == 0
    per_w = batch // _NW
    n_chunks = per_w // _CH
    rpt = n_rows // _NS
    mesh = plsc.VectorSubcoreMesh(core_axis_name="c", subcore_axis_name="s")

    @functools.partial(
        pl.kernel,
        out_type=jax.ShapeDtypeStruct((_NC * n_rows, feat), jnp.float32),
        mesh=mesh,
        scratch_types=[
            pltpu.VMEM((_CH,), jnp.int32),
            pltpu.VMEM((_CH, feat), jnp.float32),
            pltpu.VMEM_SHARED((n_rows, feat), jnp.float32),
        ],
    )
    def sk(upd, idx, zeros, out, idx_v, rows_v, acc):
        c = lax.axis_index("c")
        s = lax.axis_index("s")

        def zcopy(q, carry):
            pltpu.sync_copy(zeros, acc.at[pl.ds(s * rpt + q * _CH, _CH)])
            return carry

        lax.fori_loop(0, rpt // _CH, zcopy, 0)
        plsc.subcore_barrier()

        base = pl.multiple_of((c * _NS + s) * per_w, _CH)

        def body(j, carry):
            off = pl.multiple_of(base + j * _CH, _CH)
            pltpu.sync_copy(idx.at[pl.ds(off, _CH)], idx_v)
            pltpu.sync_copy(upd.at[pl.ds(off, _CH)], rows_v)
            pltpu.sync_copy(rows_v, acc.at[idx_v], add=True)
            return carry

        lax.fori_loop(0, n_chunks, body, 0)
        plsc.subcore_barrier()
        dst_row = pl.multiple_of(c * n_rows + s * rpt, _CH)
        pltpu.sync_copy(acc.at[pl.ds(s * rpt, rpt)], out.at[pl.ds(dst_row, rpt)])

    return sk


_NROWS_PAD = _NS * _CH


@jax.custom_vjp
def _gather(table, idx):
    return _make_gather(table.shape[0], table.shape[1], idx.shape[0])(table, idx)


def _gather_fwd(table, idx):
    return _gather(table, idx), (idx, table.shape[0])


def _gather_bwd(res, ct):
    idx, n = res
    return _scatter(ct, idx, n), None


@functools.partial(jax.custom_vjp, nondiff_argnums=(2,))
def _scatter(upd, idx, n):
    assert n % _NROWS_PAD == 0
    zeros = jnp.zeros((_CH, upd.shape[1]), jnp.float32)
    parts = _make_scatter(n, upd.shape[1], upd.shape[0])(upd, idx, zeros)
    return parts[:n] + parts[n:]


def _scatter_fwd(upd, idx, n):
    return _scatter(upd, idx, n), idx


def _scatter_bwd(n, res, ct):
    idx = res
    return _gather(ct, idx), None


_gather.defvjp(_gather_fwd, _gather_bwd)
_scatter.defvjp(_scatter_fwd, _scatter_bwd)


def kernel(at_no, coord, edge_index, charge, spin, embed_table, atom_sp, mw1, mb1, mw2, mb2, fw, fb, uu, uv, uw1, ub1, uw2, ub2, ow1, ob1, ow2, ob2):
    N, D = coord.shape[0], embed_table.shape[1]
    E = edge_index.shape[1]
    L = mw1.shape[0]
    nelem = embed_table.shape[0]
    src, dst = edge_index[0], edge_index[1]

    quant = _NW * _CH * 2
    Ep = ((E + quant - 1) // quant) * quant
    Np = ((N + quant - 1) // quant) * quant
    src_p = jnp.concatenate([src, jnp.zeros((Ep - E,), src.dtype)])
    dst_p = jnp.concatenate([dst, jnp.zeros((Ep - E,), dst.dtype)])
    at_no_p = jnp.concatenate([at_no, jnp.zeros((Np - N,), at_no.dtype)])
    emask = (jnp.arange(Ep) < E).astype(jnp.float32)

    NP = ((N + _NROWS_PAD - 1) // _NROWS_PAD) * _NROWS_PAD

    nelem_p = ((nelem + 127) // 128) * 128
    tbl = jnp.concatenate(
        [embed_table, atom_sp[:, None], jnp.zeros((nelem, 127), jnp.float32)], axis=1)
    tbl = jnp.pad(tbl, ((0, nelem_p - nelem), (0, 0)))
    g0 = _gather(tbl, at_no_p)
    x_s0 = jnp.pad(g0[:N, :D], ((0, NP - N), (0, 0)))
    sp_sum = jnp.sum(g0[:N, D])

    coordp0 = jnp.pad(coord, ((0, NP - N), (0, 125)))
    sd_p = jnp.concatenate([src_p, dst_p])

    centers = jnp.linspace(0.0, _CUTOFF, _NB)
    gamma = (_NB / _CUTOFF) ** 2

    def efn(coordp):
        cg = _gather(coordp, sd_p)
        vec = cg[:Ep, :3] - cg[Ep:, :3]
        dist = jnp.sqrt(jnp.sum(vec * vec, axis=-1) + 1e-9)
        dirv = vec / dist[:, None]
        rbf = jnp.exp(-gamma * (dist[:, None] - centers[None, :]) ** 2)
        fcut = 0.5 * (jnp.cos(jnp.pi * jnp.clip(dist, 0.0, _CUTOFF) / _CUTOFF) + 1.0) * emask
        x_s = x_s0
        xv = [jnp.zeros((NP, D), jnp.float32) for _ in range(3)]
        for l in range(L):
            phi = jax.nn.silu(x_s @ mw1[l] + mb1[l]) @ mw2[l] + mb2[l]
            Wf = (rbf @ fw[l] + fb[l]) * fcut[:, None]
            p0 = _gather(phi[:, :D], src_p)
            p1 = _gather(phi[:, D:2 * D], src_p)
            p2 = _gather(phi[:, 2 * D:], src_p)
            ms = p0 * Wf[:, :D]
            mv1 = p1 * Wf[:, D:2 * D]
            mv2 = p2 * Wf[:, 2 * D:]
            x_s = x_s + _scatter(ms, dst_p, NP)
            xv = [xv[k] + _scatter(_gather(xv[k], src_p) * mv1 + dirv[:, k:k + 1] * mv2, dst_p, NP)
                  for k in range(3)]
            xvs = jnp.stack(xv, axis=1)
            Uv = xvs @ uu[l]
            Vv = xvs @ uv[l]
            Vn = jnp.sqrt(jnp.sum(Vv * Vv, axis=1) + 1e-9)
            a = jax.nn.silu(jnp.concatenate([x_s, Vn], axis=-1) @ uw1[l] + ub1[l]) @ uw2[l] + ub2[l]
            avv, asv, ass = jnp.split(a, 3, axis=-1)
            xvs = xvs + avv[:, None, :] * Uv
            x_s = x_s + ass + asv * jnp.sum(Uv * Vv, axis=1)
            xv = [xvs[:, 0], xvs[:, 1], xvs[:, 2]]
        atom_out = jax.nn.silu(x_s @ ow1 + ob1) @ ow2 + ob2
        return jnp.sum(atom_out[:N]) + sp_sum

    energy, gpad = jax.value_and_grad(efn)(coordp0)
    return (energy, gpad[:N, :3])


# speedup vs baseline: 6.6776x; 1.0744x over previous
import functools

import jax
import jax.numpy as jnp
from jax import lax
from jax.experimental import pallas as pl
from jax.experimental.pallas import tpu as pltpu
from jax.experimental.pallas import tpu_sc as plsc

_NC, _NS = 2, 16
_NW = _NC * _NS
_CH = 128
_CUTOFF = 5.0
_NB = 20


@functools.lru_cache(maxsize=None)
def _make_gather(n_rows, feat, batch):
    assert batch % (_NW * _CH * 2) == 0 and feat % 128 == 0
    per_w = batch // _NW
    n_chunks = per_w // _CH
    mesh = plsc.VectorSubcoreMesh(core_axis_name="c", subcore_axis_name="s")

    @functools.partial(
        pl.kernel,
        out_type=jax.ShapeDtypeStruct((batch, feat), jnp.float32),
        mesh=mesh,
        scratch_types=[
            pltpu.VMEM((per_w,), jnp.int32),
            pltpu.VMEM((_CH, feat), jnp.float32),
            pltpu.VMEM((_CH, feat), jnp.float32),
            pltpu.SemaphoreType.DMA,
            pltpu.SemaphoreType.DMA,
        ],
    )
    def gk(table, idx, out, idx_full, rows0, rows1, sem0, sem1):
        wid = lax.axis_index("s") * _NC + lax.axis_index("c")
        base = pl.multiple_of(wid * per_w, _CH)
        pltpu.sync_copy(idx.at[pl.ds(base, per_w)], idx_full)

        def ichunk(j):
            return idx_full.at[pl.ds(j * _CH, _CH)]

        pltpu.async_copy(table.at[ichunk(0)], rows0, sem0)

        def body(p, carry):
            j0 = p * 2
            pltpu.async_copy(table.at[ichunk(j0 + 1)], rows1, sem1)
            pltpu.make_async_copy(table.at[ichunk(j0)], rows0, sem0).wait()
            pltpu.sync_copy(rows0, out.at[pl.ds(base + j0 * _CH, _CH)])

            @pl.when(p + 1 < n_chunks // 2)
            def _():
                pltpu.async_copy(table.at[ichunk(j0 + 2)], rows0, sem0)

            pltpu.make_async_copy(table.at[ichunk(j0 + 1)], rows1, sem1).wait()
            pltpu.sync_copy(rows1, out.at[pl.ds(base + (j0 + 1) * _CH, _CH)])
            return carry

        lax.fori_loop(0, n_chunks // 2, body, 0)

    return gk


@functools.lru_cache(maxsize=None)
def _make_scatter(n_rows, feat, batch):
    assert batch % (_NW * _CH * 2) == 0 and feat % 128 == 0
    assert n_rows % (_NS * _CH) == 0
    per_w = batch // _NW
    n_chunks = per_w // _CH
    rpt = n_rows // _NS
    mesh = plsc.VectorSubcoreMesh(core_axis_name="c", subcore_axis_name="s")

    @functools.partial(
        pl.kernel,
        out_type=jax.ShapeDtypeStruct((_NC * n_rows, feat), jnp.float32),
        mesh=mesh,
        scratch_types=[
            pltpu.VMEM((_CH,), jnp.int32),
            pltpu.VMEM((_CH,), jnp.int32),
            pltpu.VMEM((_CH, feat), jnp.float32),
            pltpu.VMEM((_CH, feat), jnp.float32),
            pltpu.VMEM_SHARED((n_rows, feat), jnp.float32),
            pltpu.SemaphoreType.DMA,
            pltpu.SemaphoreType.DMA,
        ],
    )
    def sk(upd, idx, zeros, out, idx0, idx1, rows0, rows1, acc, sem0, sem1):
        c = lax.axis_index("c")
        s = lax.axis_index("s")

        def zcopy(q, carry):
            pltpu.sync_copy(zeros, acc.at[pl.ds(s * rpt + q * _CH, _CH)])
            return carry

        lax.fori_loop(0, rpt // _CH, zcopy, 0)
        plsc.subcore_barrier()

        base = pl.multiple_of((c * _NS + s) * per_w, _CH)

        def load(j, idx_v, rows_v, sem):
            off = pl.multiple_of(base + j * _CH, _CH)
            pltpu.async_copy(idx.at[pl.ds(off, _CH)], idx_v, sem)
            pltpu.async_copy(upd.at[pl.ds(off, _CH)], rows_v, sem)

        def drain(j, idx_v, rows_v, sem):
            off = pl.multiple_of(base + j * _CH, _CH)
            pltpu.make_async_copy(idx.at[pl.ds(off, _CH)], idx_v, sem).wait()
            pltpu.make_async_copy(upd.at[pl.ds(off, _CH)], rows_v, sem).wait()

        load(0, idx0, rows0, sem0)

        def body(p, carry):
            j0 = p * 2
            load(j0 + 1, idx1, rows1, sem1)
            drain(j0, idx0, rows0, sem0)
            pltpu.sync_copy(rows0, acc.at[idx0], add=True)

            @pl.when(p + 1 < n_chunks // 2)
            def _():
                load(j0 + 2, idx0, rows0, sem0)

            drain(j0 + 1, idx1, rows1, sem1)
            pltpu.sync_copy(rows1, acc.at[idx1], add=True)
            return carry

        lax.fori_loop(0, n_chunks // 2, body, 0)
        plsc.subcore_barrier()
        dst_row = pl.multiple_of(c * n_rows + s * rpt, _CH)
        pltpu.sync_copy(acc.at[pl.ds(s * rpt, rpt)], out.at[pl.ds(dst_row, rpt)])

    return sk


_NROWS_PAD = _NS * _CH


@jax.custom_vjp
def _gather(table, idx):
    return _make_gather(table.shape[0], table.shape[1], idx.shape[0])(table, idx)


def _gather_fwd(table, idx):
    return _gather(table, idx), (idx, table.shape[0])


def _gather_bwd(res, ct):
    idx, n = res
    return _scatter(ct, idx, n), None


@functools.partial(jax.custom_vjp, nondiff_argnums=(2,))
def _scatter(upd, idx, n):
    assert n % _NROWS_PAD == 0
    zeros = jnp.zeros((_CH, upd.shape[1]), jnp.float32)
    parts = _make_scatter(n, upd.shape[1], upd.shape[0])(upd, idx, zeros)
    return parts[:n] + parts[n:]


def _scatter_fwd(upd, idx, n):
    return _scatter(upd, idx, n), idx


def _scatter_bwd(n, res, ct):
    idx = res
    return _gather(ct, idx), None


_gather.defvjp(_gather_fwd, _gather_bwd)
_scatter.defvjp(_scatter_fwd, _scatter_bwd)


def kernel(at_no, coord, edge_index, charge, spin, embed_table, atom_sp, mw1, mb1, mw2, mb2, fw, fb, uu, uv, uw1, ub1, uw2, ub2, ow1, ob1, ow2, ob2):
    N, D = coord.shape[0], embed_table.shape[1]
    E = edge_index.shape[1]
    L = mw1.shape[0]
    nelem = embed_table.shape[0]
    src, dst = edge_index[0], edge_index[1]

    quant = _NW * _CH * 2
    Ep = ((E + quant - 1) // quant) * quant
    Np = ((N + quant - 1) // quant) * quant
    src_p = jnp.concatenate([src, jnp.zeros((Ep - E,), src.dtype)])
    dst_p = jnp.concatenate([dst, jnp.zeros((Ep - E,), dst.dtype)])
    at_no_p = jnp.concatenate([at_no, jnp.zeros((Np - N,), at_no.dtype)])
    emask = (jnp.arange(Ep) < E).astype(jnp.float32)

    NP = ((N + _NROWS_PAD - 1) // _NROWS_PAD) * _NROWS_PAD

    nelem_p = ((nelem + 127) // 128) * 128
    tbl = jnp.concatenate(
        [embed_table, atom_sp[:, None], jnp.zeros((nelem, 127), jnp.float32)], axis=1)
    tbl = jnp.pad(tbl, ((0, nelem_p - nelem), (0, 0)))
    g0 = _gather(tbl, at_no_p)
    x_s0 = jnp.pad(g0[:N, :D], ((0, NP - N), (0, 0)))
    sp_sum = jnp.sum(g0[:N, D])

    coordp0 = jnp.pad(coord, ((0, NP - N), (0, 125)))
    sd_p = jnp.concatenate([src_p, dst_p])

    centers = jnp.linspace(0.0, _CUTOFF, _NB)
    gamma = (_NB / _CUTOFF) ** 2

    def efn(coordp):
        cg = _gather(coordp, sd_p)
        vec = cg[:Ep, :3] - cg[Ep:, :3]
        dist = jnp.sqrt(jnp.sum(vec * vec, axis=-1) + 1e-9)
        dirv = vec / dist[:, None]
        rbf = jnp.exp(-gamma * (dist[:, None] - centers[None, :]) ** 2)
        fcut = 0.5 * (jnp.cos(jnp.pi * jnp.clip(dist, 0.0, _CUTOFF) / _CUTOFF) + 1.0) * emask
        x_s = x_s0
        xv = [jnp.zeros((NP, D), jnp.float32) for _ in range(3)]
        for l in range(L):
            phi = jax.nn.silu(x_s @ mw1[l] + mb1[l]) @ mw2[l] + mb2[l]
            Wf = (rbf @ fw[l] + fb[l]) * fcut[:, None]
            p0 = _gather(phi[:, :D], src_p)
            p1 = _gather(phi[:, D:2 * D], src_p)
            p2 = _gather(phi[:, 2 * D:], src_p)
            ms = p0 * Wf[:, :D]
            mv1 = p1 * Wf[:, D:2 * D]
            mv2 = p2 * Wf[:, 2 * D:]
            x_s = x_s + _scatter(ms, dst_p, NP)
            xv = [xv[k] + _scatter(_gather(xv[k], src_p) * mv1 + dirv[:, k:k + 1] * mv2, dst_p, NP)
                  for k in range(3)]
            xvs = jnp.stack(xv, axis=1)
            Uv = xvs @ uu[l]
            Vv = xvs @ uv[l]
            Vn = jnp.sqrt(jnp.sum(Vv * Vv, axis=1) + 1e-9)
            a = jax.nn.silu(jnp.concatenate([x_s, Vn], axis=-1) @ uw1[l] + ub1[l]) @ uw2[l] + ub2[l]
            avv, asv, ass = jnp.split(a, 3, axis=-1)
            xvs = xvs + avv[:, None, :] * Uv
            x_s = x_s + ass + asv * jnp.sum(Uv * Vv, axis=1)
            xv = [xvs[:, 0], xvs[:, 1], xvs[:, 2]]
        atom_out = jax.nn.silu(x_s @ ow1 + ob1) @ ow2 + ob2
        return jnp.sum(atom_out[:N]) + sp_sum

    energy, gpad = jax.value_and_grad(efn)(coordp0)
    return (energy, gpad[:N, :3])
